# baseline (speedup 1.0000x reference)
import functools

import jax
import jax.numpy as jnp
from jax.experimental import pallas as pl
from jax.experimental.pallas import tpu as pltpu

_NUM_LEVELS = 4
_K = 1024
_D = 256
_N = 8192
_B = 512


def _vq_kernel(x_ref, cbsq_ref,
               cb0, cb1, cb2, cb3,
               h0, h1, h2, h3,
               m0, m1, m2, m3,
               l0, l1, l2, l3,
               idx_ref, r_ref, e_ref, z_ref, cnt_ref,
               hist_ref):
    i = pl.program_id(0)
    nb = pl.num_programs(0)

    @pl.when(i == 0)
    def _init():
        hist_ref[...] = jnp.zeros_like(hist_ref)

    cbs = (cb0, cb1, cb2, cb3)
    his = (h0, h1, h2, h3)
    mis = (m0, m1, m2, m3)
    los = (l0, l1, l2, l3)

    x0 = x_ref[...]
    xcur = x0
    ids = jax.lax.broadcasted_iota(jnp.int32, (_B, _K), 1)
    qsum = None
    for l in range(_NUM_LEVELS):
        cb = cbs[l][...]
        xc2 = jax.lax.dot_general(
            -2.0 * xcur, cb, (((1,), (1,)), ((), ())),
            preferred_element_type=jnp.float32)
        xx = jnp.sum(xcur * xcur, axis=1, keepdims=True)
        d2 = (xx + xc2) + cbsq_ref[l, :][None, :]
        m = jnp.min(d2, axis=1, keepdims=True)
        thr = m + m * 1e-6
        mask = d2 <= thr
        ncand = jnp.sum(mask.astype(jnp.int32))
        ok = jnp.logical_and(ncand == _B, jnp.min(m) > 0.0)

        def _fast(d2=d2, mask=mask):
            return jnp.min(jnp.where(mask, ids, _K), axis=1)

        def _slow(d2=d2):
            dist = jnp.sqrt(jnp.maximum(d2, 0.0))
            mind = jnp.min(dist, axis=1, keepdims=True)
            return jnp.min(jnp.where(dist == mind, ids, _K), axis=1)

        idx = jax.lax.cond(ok, _fast, _slow)
        onehot = (ids == idx[:, None]).astype(jnp.bfloat16)
        qhi = jax.lax.dot_general(
            onehot, his[l][...], (((1,), (0,)), ((), ())),
            preferred_element_type=jnp.float32)
        qmi = jax.lax.dot_general(
            onehot, mis[l][...], (((1,), (0,)), ((), ())),
            preferred_element_type=jnp.float32)
        qlo = jax.lax.dot_general(
            onehot, los[l][...], (((1,), (0,)), ((), ())),
            preferred_element_type=jnp.float32)
        q = (qhi + qmi) + qlo
        idx_ref[:, l:l + 1] = idx[:, None]
        r_ref[:, l, :] = xcur
        e_ref[:, l, :] = q
        colsum = jnp.sum(onehot.astype(jnp.float32), axis=0, keepdims=True)
        hist_ref[l:l + 1, :] += colsum
        qsum = q if qsum is None else qsum + q
        xcur = xcur - q
    z_ref[...] = qsum

    @pl.when(i == nb - 1)
    def _finish():
        used = hist_ref[0:_NUM_LEVELS, :]
        cnt_ref[...] = jnp.sum((used < 1.0).astype(jnp.int32),
                               axis=(0, 1), keepdims=True)


@functools.partial(jax.jit, static_argnames=())
def kernel(x, cb0, cb1, cb2, cb3):
    cbs = [cb0, cb1, cb2, cb3]
    cbsq = jnp.stack([jnp.sum(cb * cb, axis=1) for cb in cbs], axis=0)
    his, mis, los = [], [], []
    for cb in cbs:
        hi = jax.lax.optimization_barrier(cb.astype(jnp.bfloat16))
        hi32 = jax.lax.optimization_barrier(hi.astype(jnp.float32))
        mid = jax.lax.optimization_barrier((cb - hi32).astype(jnp.bfloat16))
        mid32 = jax.lax.optimization_barrier(mid.astype(jnp.float32))
        lo = (cb - hi32 - mid32).astype(jnp.bfloat16)
        his.append(hi)
        mis.append(mid)
        los.append(lo)

    nb = _N // _B
    full = lambda i: (0, 0)
    in_specs = [
            pl.BlockSpec((_B, _D), lambda i: (i, 0)),
            pl.BlockSpec((_NUM_LEVELS, _K), full),
    ] + [pl.BlockSpec((_K, _D), full)] * 16
    out_specs = [
        pl.BlockSpec((_B, _NUM_LEVELS), lambda i: (i, 0)),
        pl.BlockSpec((_B, _NUM_LEVELS, _D), lambda i: (i, 0, 0)),
        pl.BlockSpec((_B, _NUM_LEVELS, _D), lambda i: (i, 0, 0)),
        pl.BlockSpec((_B, _D), lambda i: (i, 0)),
        pl.BlockSpec((1, 1), full),
    ]
    out_shapes = [
        jax.ShapeDtypeStruct((_N, _NUM_LEVELS), jnp.int32),
        jax.ShapeDtypeStruct((_N, _NUM_LEVELS, _D), jnp.float32),
        jax.ShapeDtypeStruct((_N, _NUM_LEVELS, _D), jnp.float32),
        jax.ShapeDtypeStruct((_N, _D), jnp.float32),
        jax.ShapeDtypeStruct((1, 1), jnp.int32),
    ]
    idx, r_s, e_s, z_hat, cnt = pl.pallas_call(
        _vq_kernel,
        grid=(nb,),
        in_specs=in_specs,
        out_specs=out_specs,
        out_shape=out_shapes,
        scratch_shapes=[pltpu.VMEM((8, _K), jnp.float32)],
        compiler_params=pltpu.CompilerParams(
            dimension_semantics=("arbitrary",),
        ),
    )(x, cbsq, *cbs, *his, *mis, *los)
    return (idx.astype(jnp.int64), r_s, e_s, z_hat,
            jnp.reshape(cnt, ()))


# speedup vs baseline: 1.0785x; 1.0785x over previous
import functools

import jax
import jax.numpy as jnp
from jax.experimental import pallas as pl
from jax.experimental.pallas import tpu as pltpu

_NUM_LEVELS = 4
_K = 1024
_D = 256
_N = 8192
_B = 512


def _vq_kernel(x_ref, cbsq_ref,
               cb0, cb1, cb2, cb3,
               h0, h1, h2, h3,
               m0, m1, m2, m3,
               l0, l1, l2, l3,
               idx_ref, r_ref, e_ref, z_ref, cnt_ref,
               hist_ref):
    i = pl.program_id(0)
    nb = pl.num_programs(0)

    @pl.when(i == 0)
    def _init():
        hist_ref[...] = jnp.zeros_like(hist_ref)

    cbs = (cb0, cb1, cb2, cb3)
    his = (h0, h1, h2, h3)
    mis = (m0, m1, m2, m3)
    los = (l0, l1, l2, l3)

    x0 = x_ref[...]
    xcur = x0
    ids = jax.lax.broadcasted_iota(jnp.int32, (_B, _K), 1)
    qsum = None
    for l in range(_NUM_LEVELS):
        cb = cbs[l][...]
        xc2 = jax.lax.dot_general(
            -2.0 * xcur, cb, (((1,), (1,)), ((), ())),
            preferred_element_type=jnp.float32)
        xx = jnp.sum(xcur * xcur, axis=1, keepdims=True)
        d2 = (xx + xc2) + cbsq_ref[l, :][None, :]
        dist = jnp.sqrt(jnp.maximum(d2, 0.0))
        idx = jnp.argmin(dist, axis=1).astype(jnp.int32)
        onehot = (ids == idx[:, None]).astype(jnp.bfloat16)
        qhi = jax.lax.dot_general(
            onehot, his[l][...], (((1,), (0,)), ((), ())),
            preferred_element_type=jnp.float32)
        qmi = jax.lax.dot_general(
            onehot, mis[l][...], (((1,), (0,)), ((), ())),
            preferred_element_type=jnp.float32)
        qlo = jax.lax.dot_general(
            onehot, los[l][...], (((1,), (0,)), ((), ())),
            preferred_element_type=jnp.float32)
        q = (qhi + qmi) + qlo
        idx_ref[:, l:l + 1] = idx[:, None]
        r_ref[:, l, :] = xcur
        e_ref[:, l, :] = q
        colsum = jnp.sum(onehot.astype(jnp.float32), axis=0, keepdims=True)
        hist_ref[l:l + 1, :] += colsum
        qsum = q if qsum is None else qsum + q
        xcur = xcur - q
    z_ref[...] = qsum

    @pl.when(i == nb - 1)
    def _finish():
        used = hist_ref[0:_NUM_LEVELS, :]
        cnt_ref[...] = jnp.sum((used < 1.0).astype(jnp.int32),
                               axis=(0, 1), keepdims=True)


@functools.partial(jax.jit, static_argnames=())
def kernel(x, cb0, cb1, cb2, cb3):
    cbs = [cb0, cb1, cb2, cb3]
    cbsq = jnp.stack([jnp.sum(cb * cb, axis=1) for cb in cbs], axis=0)
    his, mis, los = [], [], []
    for cb in cbs:
        hi = jax.lax.optimization_barrier(cb.astype(jnp.bfloat16))
        hi32 = jax.lax.optimization_barrier(hi.astype(jnp.float32))
        mid = jax.lax.optimization_barrier((cb - hi32).astype(jnp.bfloat16))
        mid32 = jax.lax.optimization_barrier(mid.astype(jnp.float32))
        lo = (cb - hi32 - mid32).astype(jnp.bfloat16)
        his.append(hi)
        mis.append(mid)
        los.append(lo)

    nb = _N // _B
    full = lambda i: (0, 0)
    in_specs = [
            pl.BlockSpec((_B, _D), lambda i: (i, 0)),
            pl.BlockSpec((_NUM_LEVELS, _K), full),
    ] + [pl.BlockSpec((_K, _D), full)] * 16
    out_specs = [
        pl.BlockSpec((_B, _NUM_LEVELS), lambda i: (i, 0)),
        pl.BlockSpec((_B, _NUM_LEVELS, _D), lambda i: (i, 0, 0)),
        pl.BlockSpec((_B, _NUM_LEVELS, _D), lambda i: (i, 0, 0)),
        pl.BlockSpec((_B, _D), lambda i: (i, 0)),
        pl.BlockSpec((1, 1), full),
    ]
    out_shapes = [
        jax.ShapeDtypeStruct((_N, _NUM_LEVELS), jnp.int32),
        jax.ShapeDtypeStruct((_N, _NUM_LEVELS, _D), jnp.float32),
        jax.ShapeDtypeStruct((_N, _NUM_LEVELS, _D), jnp.float32),
        jax.ShapeDtypeStruct((_N, _D), jnp.float32),
        jax.ShapeDtypeStruct((1, 1), jnp.int32),
    ]
    idx, r_s, e_s, z_hat, cnt = pl.pallas_call(
        _vq_kernel,
        grid=(nb,),
        in_specs=in_specs,
        out_specs=out_specs,
        out_shape=out_shapes,
        scratch_shapes=[pltpu.VMEM((8, _K), jnp.float32)],
        compiler_params=pltpu.CompilerParams(
            dimension_semantics=("arbitrary",),
        ),
    )(x, cbsq, *cbs, *his, *mis, *los)
    return (idx.astype(jnp.int64), r_s, e_s, z_hat,
            jnp.reshape(cnt, ()))


# speedup vs baseline: 1.1304x; 1.0481x over previous
import functools

import jax
import jax.numpy as jnp
from jax.experimental import pallas as pl
from jax.experimental.pallas import tpu as pltpu

_NUM_LEVELS = 4
_K = 1024
_D = 256
_N = 8192
_B = 512


def _vq_kernel(x_ref, cbsq_ref,
               cb0, cb1, cb2, cb3,
               s0, s1, s2, s3,
               idx_ref, r_ref, e_ref, z_ref, cnt_ref,
               hist_ref):
    i = pl.program_id(0)
    nb = pl.num_programs(0)

    @pl.when(i == 0)
    def _init():
        hist_ref[...] = jnp.zeros_like(hist_ref)

    cbs = (cb0, cb1, cb2, cb3)
    split_ref = (s0, s1, s2, s3)

    x0 = x_ref[...]
    xcur = x0
    ids = jax.lax.broadcasted_iota(jnp.int32, (_B, _K), 1)
    ones8 = jnp.ones((8, _B), dtype=jnp.bfloat16)
    qsum = None
    for l in range(_NUM_LEVELS):
        cb = cbs[l][...]
        xc2 = jax.lax.dot_general(
            -2.0 * xcur, cb, (((1,), (1,)), ((), ())),
            preferred_element_type=jnp.float32)
        xx = jnp.sum(xcur * xcur, axis=1, keepdims=True)
        d2 = (xx + xc2) + cbsq_ref[l, :][None, :]
        m = jnp.min(d2, axis=1, keepdims=True)
        mpos = m > 0.0
        s = jnp.sqrt(jnp.maximum(m, 0.0))
        mbits = jax.lax.bitcast_convert_type(m, jnp.int32)
        mk = jax.lax.bitcast_convert_type(
            mbits + jax.lax.broadcasted_iota(jnp.int32, (_B, 8), 1),
            jnp.float32)
        tie = jnp.sqrt(mk) == s
        u = jnp.max(jnp.where(tie, mk, m), axis=1, keepdims=True)
        u = jnp.where(mpos, u, 0.0)
        idx = jnp.min(jnp.where(d2 <= u, ids, _K), axis=1)
        ohb = ids == idx[:, None]
        onehot = ohb.astype(jnp.bfloat16)
        q3 = jax.lax.dot_general(
            onehot, split_ref[l][...], (((1,), (0,)), ((), ())),
            preferred_element_type=jnp.float32)
        q = (q3[:, :_D] + q3[:, _D:2 * _D]) + q3[:, 2 * _D:]
        idx_ref[:, l:l + 1] = idx[:, None]
        r_ref[:, l, :] = xcur
        e_ref[:, l, :] = q
        colsum = jax.lax.dot_general(
            ones8, onehot, (((1,), (0,)), ((), ())),
            preferred_element_type=jnp.float32)
        hist_ref[l:l + 1, :] += colsum[0:1, :]
        qsum = q if qsum is None else qsum + q
        xcur = xcur - q
    z_ref[...] = qsum

    @pl.when(i == nb - 1)
    def _finish():
        used = hist_ref[0:_NUM_LEVELS, :]
        cnt_ref[...] = jnp.sum((used < 1.0).astype(jnp.int32),
                               axis=(0, 1), keepdims=True)


@functools.partial(jax.jit, static_argnames=())
def kernel(x, cb0, cb1, cb2, cb3):
    cbs = [cb0, cb1, cb2, cb3]
    cbsq = jnp.stack([jnp.sum(cb * cb, axis=1) for cb in cbs], axis=0)
    splits = []
    for cb in cbs:
        hi = jax.lax.optimization_barrier(cb.astype(jnp.bfloat16))
        hi32 = jax.lax.optimization_barrier(hi.astype(jnp.float32))
        mid = jax.lax.optimization_barrier((cb - hi32).astype(jnp.bfloat16))
        mid32 = jax.lax.optimization_barrier(mid.astype(jnp.float32))
        lo = (cb - hi32 - mid32).astype(jnp.bfloat16)
        splits.append(jnp.concatenate([hi, mid, lo], axis=1))

    nb = _N // _B
    full = lambda i: (0, 0)
    in_specs = [
            pl.BlockSpec((_B, _D), lambda i: (i, 0)),
            pl.BlockSpec((_NUM_LEVELS, _K), full),
    ] + [pl.BlockSpec((_K, _D), full)] * 4 \
      + [pl.BlockSpec((_K, 3 * _D), full)] * 4
    out_specs = [
        pl.BlockSpec((_B, _NUM_LEVELS), lambda i: (i, 0)),
        pl.BlockSpec((_B, _NUM_LEVELS, _D), lambda i: (i, 0, 0)),
        pl.BlockSpec((_B, _NUM_LEVELS, _D), lambda i: (i, 0, 0)),
        pl.BlockSpec((_B, _D), lambda i: (i, 0)),
        pl.BlockSpec((1, 1), full),
    ]
    out_shapes = [
        jax.ShapeDtypeStruct((_N, _NUM_LEVELS), jnp.int32),
        jax.ShapeDtypeStruct((_N, _NUM_LEVELS, _D), jnp.float32),
        jax.ShapeDtypeStruct((_N, _NUM_LEVELS, _D), jnp.float32),
        jax.ShapeDtypeStruct((_N, _D), jnp.float32),
        jax.ShapeDtypeStruct((1, 1), jnp.int32),
    ]
    idx, r_s, e_s, z_hat, cnt = pl.pallas_call(
        _vq_kernel,
        grid=(nb,),
        in_specs=in_specs,
        out_specs=out_specs,
        out_shape=out_shapes,
        scratch_shapes=[pltpu.VMEM((8, _K), jnp.float32)],
        compiler_params=pltpu.CompilerParams(
            dimension_semantics=("arbitrary",),
        ),
    )(x, cbsq, *cbs, *splits)
    return (idx.astype(jnp.int64), r_s, e_s, z_hat,
            jnp.reshape(cnt, ()))
